# baseline (device time: 96414 ns/iter reference)
import jax
import jax.numpy as jnp
from jax import lax
from jax.experimental import pallas as pl
from jax.experimental.pallas import tpu as pltpu

T = 2048
D = 1024
TH = T // 2


def kernel(ids, E):
    v_local = E.shape[0]
    my_x = lax.axis_index("x")
    my_y = lax.axis_index("y")

    ids_half = lax.dynamic_slice(ids, (my_x * TH,), (TH,))
    local = ids_half - my_y * v_local
    mask = (local >= 0) & (local < v_local)
    safe = jnp.where(mask, local, 0)
    partial = jnp.where(mask[:, None], jnp.take(E, safe, axis=0), 0.0)
    partial = partial.astype(jnp.float32)

    C = 8
    CH = TH // C

    def body(partial_ref, out_ref, comm_ref, sum_ref, other_ref,
             send_a, recv_a, send_b, recv_b):
        x = lax.axis_index("x")
        y = lax.axis_index("y")
        y_nbr = (x, 1 - y)
        x_nbr = (1 - x, y)

        barrier = pltpu.get_barrier_semaphore()
        for nbr in (y_nbr, x_nbr):
            pl.semaphore_signal(
                barrier, inc=1, device_id=nbr,
                device_id_type=pl.DeviceIdType.MESH,
            )
        pl.semaphore_wait(barrier, 2)

        def rdma_a(c):
            return pltpu.make_async_remote_copy(
                src_ref=partial_ref.at[pl.ds(c * CH, CH), :],
                dst_ref=comm_ref.at[pl.ds(c * CH, CH), :],
                send_sem=send_a.at[c],
                recv_sem=recv_a.at[c],
                device_id=y_nbr,
                device_id_type=pl.DeviceIdType.MESH,
            )

        def rdma_b(c):
            return pltpu.make_async_remote_copy(
                src_ref=sum_ref.at[pl.ds(c * CH, CH), :],
                dst_ref=other_ref.at[pl.ds(c * CH, CH), :],
                send_sem=send_b.at[c],
                recv_sem=recv_b.at[c],
                device_id=x_nbr,
                device_id_type=pl.DeviceIdType.MESH,
            )

        for c in range(C):
            rdma_a(c).start()

        for c in range(C):
            rdma_a(c).wait_recv()
            sum_ref[pl.ds(c * CH, CH), :] = (
                partial_ref[pl.ds(c * CH, CH), :]
                + comm_ref[pl.ds(c * CH, CH), :]
            )
            rdma_b(c).start()

        for c in range(C):
            rdma_a(c).wait_send()
            rdma_b(c).wait()

        out_ref[pl.ds(x * TH, TH), :] = sum_ref[:, :]
        out_ref[pl.ds((1 - x) * TH, TH), :] = other_ref[:, :]

    return pl.pallas_call(
        body,
        out_shape=jax.ShapeDtypeStruct((T, D), jnp.float32),
        in_specs=[pl.BlockSpec(memory_space=pltpu.VMEM)],
        out_specs=pl.BlockSpec(memory_space=pltpu.VMEM),
        scratch_shapes=[
            pltpu.VMEM((TH, D), jnp.float32),
            pltpu.VMEM((TH, D), jnp.float32),
            pltpu.VMEM((TH, D), jnp.float32),
            pltpu.SemaphoreType.DMA((C,)),
            pltpu.SemaphoreType.DMA((C,)),
            pltpu.SemaphoreType.DMA((C,)),
            pltpu.SemaphoreType.DMA((C,)),
        ],
        compiler_params=pltpu.CompilerParams(collective_id=0),
    )(partial)


# device time: 92782 ns/iter; 1.0391x vs baseline; 1.0391x over previous
import jax
import jax.numpy as jnp
from jax import lax
from jax.experimental import pallas as pl
from jax.experimental.pallas import tpu as pltpu

T = 2048
D = 1024
TH = T // 2


def kernel(ids, E):
    v_local = E.shape[0]
    my_x = lax.axis_index("x")
    my_y = lax.axis_index("y")

    ids_half = lax.dynamic_slice(ids, (my_x * TH,), (TH,))
    local = ids_half - my_y * v_local
    partial = jnp.take(E, local, axis=0, mode="fill", fill_value=0.0)
    partial = partial.astype(jnp.float32)

    C = 16
    CH = TH // C

    def body(partial_ref, out_ref, comm_ref, sum_ref, other_ref,
             send_a, recv_a, send_b, recv_b):
        x = lax.axis_index("x")
        y = lax.axis_index("y")
        y_nbr = (x, 1 - y)
        x_nbr = (1 - x, y)

        barrier = pltpu.get_barrier_semaphore()
        for nbr in (y_nbr, x_nbr):
            pl.semaphore_signal(
                barrier, inc=1, device_id=nbr,
                device_id_type=pl.DeviceIdType.MESH,
            )
        pl.semaphore_wait(barrier, 2)

        def rdma_a(c):
            return pltpu.make_async_remote_copy(
                src_ref=partial_ref.at[pl.ds(c * CH, CH), :],
                dst_ref=comm_ref.at[pl.ds(c * CH, CH), :],
                send_sem=send_a.at[c],
                recv_sem=recv_a.at[c],
                device_id=y_nbr,
                device_id_type=pl.DeviceIdType.MESH,
            )

        def rdma_b(c):
            return pltpu.make_async_remote_copy(
                src_ref=sum_ref.at[pl.ds(c * CH, CH), :],
                dst_ref=other_ref.at[pl.ds(c * CH, CH), :],
                send_sem=send_b.at[c],
                recv_sem=recv_b.at[c],
                device_id=x_nbr,
                device_id_type=pl.DeviceIdType.MESH,
            )

        for c in range(C):
            rdma_a(c).start()

        for c in range(C):
            rdma_a(c).wait_recv()
            sum_ref[pl.ds(c * CH, CH), :] = (
                partial_ref[pl.ds(c * CH, CH), :]
                + comm_ref[pl.ds(c * CH, CH), :]
            )
            rdma_b(c).start()

        for c in range(C):
            rdma_a(c).wait_send()
            rdma_b(c).wait()

        out_ref[pl.ds(x * TH, TH), :] = sum_ref[:, :]
        out_ref[pl.ds((1 - x) * TH, TH), :] = other_ref[:, :]

    return pl.pallas_call(
        body,
        out_shape=jax.ShapeDtypeStruct((T, D), jnp.float32),
        in_specs=[pl.BlockSpec(memory_space=pltpu.VMEM)],
        out_specs=pl.BlockSpec(memory_space=pltpu.VMEM),
        scratch_shapes=[
            pltpu.VMEM((TH, D), jnp.float32),
            pltpu.VMEM((TH, D), jnp.float32),
            pltpu.VMEM((TH, D), jnp.float32),
            pltpu.SemaphoreType.DMA((C,)),
            pltpu.SemaphoreType.DMA((C,)),
            pltpu.SemaphoreType.DMA((C,)),
            pltpu.SemaphoreType.DMA((C,)),
        ],
        compiler_params=pltpu.CompilerParams(collective_id=0),
    )(partial)
